# baseline (speedup 1.0000x reference)
import functools

import jax
import jax.numpy as jnp
from jax import lax
from jax.experimental import pallas as pl
from jax.experimental.pallas import tpu as pltpu
from jax.experimental.pallas import tpu_sc as plsc

_D = 1024
_B = 4 * 8192
_NC = 2
_NS = 16
_NW = _NC * _NS
_BPW = _B // _NW
_CHUNK = 32
_NCHUNK = _BPW // _CHUNK
_NBUF = 2


@functools.partial(
    pl.kernel,
    mesh=plsc.VectorSubcoreMesh(core_axis_name="c", subcore_axis_name="s"),
    out_type=jax.ShapeDtypeStruct((_B, _D), jnp.float32),
    scratch_types=[
        pltpu.VMEM((_NCHUNK, _CHUNK), jnp.int32),
        pltpu.VMEM((_NBUF, _CHUNK, _D), jnp.float32),
        pltpu.SemaphoreType.DMA,
        pltpu.SemaphoreType.DMA,
        pltpu.SemaphoreType.DMA,
    ],
)
def _emb_lookup(idx_hbm, w_hbm, out_hbm, idx_v, rows_v, gsem, ssem0, ssem1):
    wid = lax.axis_index("s") * _NC + lax.axis_index("c")
    base = wid * _BPW
    pltpu.sync_copy(idx_hbm.at[wid], idx_v)

    ssems = [ssem0, ssem1]
    scatters = [None] * _NCHUNK
    for i in range(_NCHUNK):
        b = i % _NBUF
        if i >= _NBUF:
            scatters[i - _NBUF].wait()
        g = pltpu.async_copy(w_hbm.at[idx_v.at[i]], rows_v.at[b], gsem)
        g.wait()
        s = pltpu.async_copy(
            rows_v.at[b], out_hbm.at[pl.ds(base + i * _CHUNK, _CHUNK)], ssems[b]
        )
        scatters[i] = s
    for i in range(_NCHUNK - _NBUF, _NCHUNK):
        scatters[i].wait()


def kernel(token_types, weight):
    idx = jnp.asarray(token_types, jnp.int32).reshape(_NW, _NCHUNK, _CHUNK)
    out = _emb_lookup(idx, weight)
    return out.reshape(token_types.shape + (_D,))
